# Initial kernel scaffold; baseline (speedup 1.0000x reference)
import jax
import jax.numpy as jnp
from jax.experimental import pallas as pl


def kernel(x, edge_index, edge_attr, W_edge, W_ne, b_ne, W_self, b_self):
    raise NotImplementedError("write your pallas kernel here")


# speedup vs baseline: 3.0117x; 3.0117x over previous
import functools

import jax
import jax.numpy as jnp
from jax import lax
from jax.experimental import pallas as pl
from jax.experimental.pallas import tpu as pltpu
from jax.experimental.pallas import tpu_sc as plsc

N = 10000
E = 320000
D = 128
DE = 16

NC = 2
NS = 16
EPT = E // (NC * NS)
CH = 80
NCHUNK = EPT // CH
NPAD = 10240
RPT = NPAD // NS


def _prep_body(x_ref, w1_ref, bne_ref, wedge_ref, wne2_ref, xh_ref, w2e_ref):
    i = pl.program_id(0)
    xh_ref[...] = (
        jnp.dot(x_ref[...], w1_ref[...], preferred_element_type=jnp.float32)
        + bne_ref[...][None, :]
    )

    @pl.when(i == 0)
    def _():
        w2e_ref[...] = jnp.dot(
            wedge_ref[...], wne2_ref[...], preferred_element_type=jnp.float32
        )


def _prep(x, W1, b_ne, W_edge, W2):
    nb = 5
    rb = N // nb
    return pl.pallas_call(
        _prep_body,
        grid=(nb,),
        in_specs=[
            pl.BlockSpec((rb, D), lambda i: (i, 0)),
            pl.BlockSpec((D, D), lambda i: (0, 0)),
            pl.BlockSpec((D,), lambda i: (0,)),
            pl.BlockSpec((DE, D), lambda i: (0, 0)),
            pl.BlockSpec((D, D), lambda i: (0, 0)),
        ],
        out_specs=[
            pl.BlockSpec((rb, D), lambda i: (i, 0)),
            pl.BlockSpec((DE, D), lambda i: (0, 0)),
        ],
        out_shape=[
            jax.ShapeDtypeStruct((N, D), jnp.float32),
            jax.ShapeDtypeStruct((DE, D), jnp.float32),
        ],
    )(x, W1, b_ne, W_edge, W2)


def _eh2_body(ea_ref, w2e_ref, eh2_ref):
    eh2_ref[...] = jnp.dot(
        ea_ref[...], w2e_ref[...], preferred_element_type=jnp.float32
    )


def _eh2(edge_attr, W2e):
    eb = 4000
    nb = E // eb
    return pl.pallas_call(
        _eh2_body,
        grid=(nb,),
        in_specs=[
            pl.BlockSpec((eb, DE), lambda i: (i, 0)),
            pl.BlockSpec((DE, D), lambda i: (0, 0)),
        ],
        out_specs=pl.BlockSpec((eb, D), lambda i: (i, 0)),
        out_shape=jax.ShapeDtypeStruct((E, D), jnp.float32),
    )(edge_attr, W2e)


def _sc_body(xh_hbm, eh2_hbm, src_hbm, dst_hbm, msum_hbm, deg_hbm,
             src_v, dst_v, rows_v, eh_v, ones_v, zrow_v, zdeg_v,
             msum_sh, deg_sh, sem):
    c = lax.axis_index("c")
    s = lax.axis_index("s")

    zero16 = jnp.zeros((16,), jnp.float32)
    one16 = jnp.ones((16,), jnp.float32)

    @pl.loop(0, 128)
    def _(r):
        for j in range(8):
            zrow_v[r, pl.ds(j * 16, 16)] = zero16

    @pl.loop(0, 128)
    def _(k):
        zdeg_v[pl.ds(k * 16, 16)] = zero16

    for k in range(CH // 16):
        ones_v[pl.ds(k * 16, 16)] = one16

    for t in range(5):
        pltpu.sync_copy(zrow_v, msum_sh.at[pl.ds(s * RPT + t * 128, 128)])

    @pl.when(s == 0)
    def _():
        for t in range(5):
            pltpu.sync_copy(zdeg_v, deg_sh.at[pl.ds(t * 2048, 2048)])

    plsc.subcore_barrier()

    ebase = (c * NS + s) * EPT

    @pl.loop(0, NCHUNK)
    def _(i):
        base = ebase + i * CH
        pltpu.sync_copy(src_hbm.at[pl.ds(base, CH)], src_v)
        pltpu.sync_copy(dst_hbm.at[pl.ds(base, CH)], dst_v)
        pltpu.async_copy(xh_hbm.at[src_v], rows_v, sem).wait()
        pltpu.sync_copy(eh2_hbm.at[pl.ds(base, CH)], eh_v)

        @pl.loop(0, CH)
        def _(r):
            for j in range(8):
                sl = pl.ds(j * 16, 16)
                rows_v[r, sl] = jnp.maximum(rows_v[r, sl] + eh_v[r, sl], 0.0)

        pltpu.sync_copy(rows_v, msum_sh.at[dst_v], add=True)
        pltpu.sync_copy(ones_v, deg_sh.at[dst_v], add=True)

    plsc.subcore_barrier()

    for t in range(5):
        sl = pl.ds(s * RPT + t * 128, 128)
        pltpu.sync_copy(msum_sh.at[sl], msum_hbm.at[c, sl])

    @pl.when(s == 0)
    def _():
        for t in range(5):
            pltpu.sync_copy(
                deg_sh.at[pl.ds(t * 2048, 2048)],
                deg_hbm.at[pl.ds(c * NPAD + t * 2048, 2048)],
            )


_sc_call = functools.partial(
    pl.kernel,
    out_type=(
        jax.ShapeDtypeStruct((NC, NPAD, D), jnp.float32),
        jax.ShapeDtypeStruct((NC * NPAD,), jnp.float32),
    ),
    mesh=plsc.VectorSubcoreMesh(
        core_axis_name="c", subcore_axis_name="s", num_cores=NC, num_subcores=NS
    ),
    scratch_types=[
        pltpu.VMEM((CH,), jnp.int32),
        pltpu.VMEM((CH,), jnp.int32),
        pltpu.VMEM((CH, D), jnp.float32),
        pltpu.VMEM((CH, D), jnp.float32),
        pltpu.VMEM((CH,), jnp.float32),
        pltpu.VMEM((128, D), jnp.float32),
        pltpu.VMEM((2048,), jnp.float32),
        pltpu.VMEM_SHARED((NPAD, D), jnp.float32),
        pltpu.VMEM_SHARED((NPAD,), jnp.float32),
        pltpu.SemaphoreType.DMA,
    ],
)(_sc_body)


def _comb_body(p_ref, deg_ref, x_ref, ws_ref, bs_ref, o_ref):
    ms = p_ref[0] + p_ref[1]
    d = deg_ref[0] + deg_ref[1]
    r = 1.0 / jnp.maximum(d, 1.0)
    sf = (
        jnp.dot(x_ref[...], ws_ref[...], preferred_element_type=jnp.float32)
        + bs_ref[...][None, :]
    )
    o_ref[...] = ms * r + sf


def _combine(msum, deg, x, W_self, b_self):
    return pl.pallas_call(
        _comb_body,
        out_shape=jax.ShapeDtypeStruct((N, D), jnp.float32),
    )(msum, deg, x, W_self, b_self)


def kernel(x, edge_index, edge_attr, W_edge, W_ne, b_ne, W_self, b_self):
    W1 = W_ne[:D, :]
    W2 = W_ne[D:, :]
    xh, W2e = _prep(x, W1, b_ne, W_edge, W2)
    eh2 = _eh2(edge_attr, W2e)
    msum, deg = _sc_call(xh, eh2, edge_index[0], edge_index[1])
    deg2 = deg.reshape(NC, NPAD)[:, :N].reshape(NC, N, 1)
    return _combine(msum[:, :N], deg2, x, W_self, b_self)


# speedup vs baseline: 4.8604x; 1.6138x over previous
import functools

import jax
import jax.numpy as jnp
from jax import lax
from jax.experimental import pallas as pl
from jax.experimental.pallas import tpu as pltpu
from jax.experimental.pallas import tpu_sc as plsc

N = 10000
E = 320000
D = 128
DE = 16

NC = 2
NS = 16
EPT = E // (NC * NS)
CH = 80
NCHUNK = EPT // CH
NPAD = 10240
RPT = NPAD // NS


def _prep_body(x_ref, w1_ref, bne_ref, wedge_ref, wne2_ref, xh_ref, w2e_ref):
    i = pl.program_id(0)
    xh_ref[...] = (
        jnp.dot(x_ref[...], w1_ref[...], preferred_element_type=jnp.float32)
        + bne_ref[...][None, :]
    )

    @pl.when(i == 0)
    def _():
        w2e_ref[...] = jnp.dot(
            wedge_ref[...], wne2_ref[...], preferred_element_type=jnp.float32
        )


def _prep(x, W1, b_ne, W_edge, W2):
    nb = 5
    rb = N // nb
    return pl.pallas_call(
        _prep_body,
        grid=(nb,),
        in_specs=[
            pl.BlockSpec((rb, D), lambda i: (i, 0)),
            pl.BlockSpec((D, D), lambda i: (0, 0)),
            pl.BlockSpec((D,), lambda i: (0,)),
            pl.BlockSpec((DE, D), lambda i: (0, 0)),
            pl.BlockSpec((D, D), lambda i: (0, 0)),
        ],
        out_specs=[
            pl.BlockSpec((rb, D), lambda i: (i, 0)),
            pl.BlockSpec((DE, D), lambda i: (0, 0)),
        ],
        out_shape=[
            jax.ShapeDtypeStruct((N, D), jnp.float32),
            jax.ShapeDtypeStruct((DE, D), jnp.float32),
        ],
    )(x, W1, b_ne, W_edge, W2)


def _eh2_body(ea_ref, w2e_ref, eh2_ref):
    eh2_ref[...] = jnp.dot(
        ea_ref[...], w2e_ref[...], preferred_element_type=jnp.float32
    )


def _eh2(edge_attr, W2e):
    eb = 4000
    nb = E // eb
    return pl.pallas_call(
        _eh2_body,
        grid=(nb,),
        in_specs=[
            pl.BlockSpec((eb, DE), lambda i: (i, 0)),
            pl.BlockSpec((DE, D), lambda i: (0, 0)),
        ],
        out_specs=pl.BlockSpec((eb, D), lambda i: (i, 0)),
        out_shape=jax.ShapeDtypeStruct((E, D), jnp.float32),
    )(edge_attr, W2e)


def _sc_body(xh_hbm, eh2_hbm, src_hbm, dst_hbm, msum_hbm, deg_hbm,
             src_b0, src_b1, dst_b0, dst_b1, rows_v0, rows_v1, eh_v0, eh_v1,
             ones_v, zdeg_v, msum_sh, deg_sh,
             idx_sem0, idx_sem1, in_sem0, in_sem1):
    c = lax.axis_index("c")
    s = lax.axis_index("s")
    wid = c * NS + s

    src_b = (src_b0, src_b1)
    dst_b = (dst_b0, dst_b1)
    rows_v = (rows_v0, rows_v1)
    eh_v = (eh_v0, eh_v1)
    idx_sems = (idx_sem0, idx_sem1)
    in_sems = (in_sem0, in_sem1)

    zero16 = jnp.zeros((16,), jnp.float32)
    one16 = jnp.ones((16,), jnp.float32)

    @pl.loop(0, CH)
    def _(r):
        for j in range(8):
            rows_v0[r, pl.ds(j * 16, 16)] = zero16

    @pl.loop(0, 128)
    def _(k):
        zdeg_v[pl.ds(k * 16, 16)] = zero16

    for k in range(CH // 16):
        ones_v[pl.ds(k * 16, 16)] = one16

    for t in range(8):
        pltpu.sync_copy(rows_v0, msum_sh.at[pl.ds(s * RPT + t * CH, CH)])

    @pl.when(s == 0)
    def _():
        for t in range(5):
            pltpu.sync_copy(zdeg_v, deg_sh.at[pl.ds(t * 2048, 2048)])

    plsc.subcore_barrier()

    ebase = wid * EPT

    def fire_idx(ic, b):
        base = ebase + ic * CH
        pltpu.async_copy(src_hbm.at[pl.ds(base, CH)], src_b[b], idx_sems[b])
        pltpu.async_copy(dst_hbm.at[pl.ds(base, CH)], dst_b[b], idx_sems[b])

    def wait_idx(ic, b):
        base = ebase + ic * CH
        pltpu.make_async_copy(
            src_hbm.at[pl.ds(base, CH)], src_b[b], idx_sems[b]
        ).wait()
        pltpu.make_async_copy(
            dst_hbm.at[pl.ds(base, CH)], dst_b[b], idx_sems[b]
        ).wait()

    def fire_data(ic, b):
        pltpu.async_copy(xh_hbm.at[src_b[b]], rows_v[b], in_sems[b])
        pltpu.async_copy(
            eh2_hbm.at[pl.ds(ebase + ic * CH, CH)], eh_v[b], in_sems[b]
        )

    def wait_data(ic, b):
        pltpu.make_async_copy(
            xh_hbm.at[src_b[b]], rows_v[b], in_sems[b]
        ).wait()
        pltpu.make_async_copy(
            eh2_hbm.at[pl.ds(ebase + ic * CH, CH)], eh_v[b], in_sems[b]
        ).wait()

    def compute(b):
        rv, ev = rows_v[b], eh_v[b]

        @pl.loop(0, CH)
        def _(r):
            for j in range(8):
                sl = pl.ds(j * 16, 16)
                rv[r, sl] = jnp.maximum(rv[r, sl] + ev[r, sl], 0.0)

    def process(ic, b, nb):
        @pl.when(ic + 1 < NCHUNK)
        def _():
            wait_idx(ic + 1, nb)
            fire_data(ic + 1, nb)

        wait_data(ic, b)
        compute(b)
        pltpu.sync_copy(rows_v[b], msum_sh.at[dst_b[b]], add=True)
        pltpu.sync_copy(ones_v, deg_sh.at[dst_b[b]], add=True)

        @pl.when(ic + 2 < NCHUNK)
        def _():
            fire_idx(ic + 2, b)

    fire_idx(0, 0)
    fire_idx(1, 1)
    wait_idx(0, 0)
    fire_data(0, 0)

    @pl.loop(0, NCHUNK - 1, step=2)
    def _(i):
        process(i, 0, 1)
        process(i + 1, 1, 0)

    process(NCHUNK - 1, 0, 1)

    plsc.subcore_barrier()

    for t in range(5):
        sl = pl.ds(s * RPT + t * 128, 128)
        pltpu.sync_copy(msum_sh.at[sl], msum_hbm.at[c, sl])

    @pl.when(s == 0)
    def _():
        for t in range(5):
            pltpu.sync_copy(
                deg_sh.at[pl.ds(t * 2048, 2048)],
                deg_hbm.at[pl.ds(c * NPAD + t * 2048, 2048)],
            )


_sc_call = functools.partial(
    pl.kernel,
    out_type=(
        jax.ShapeDtypeStruct((NC, NPAD, D), jnp.float32),
        jax.ShapeDtypeStruct((NC * NPAD,), jnp.float32),
    ),
    mesh=plsc.VectorSubcoreMesh(
        core_axis_name="c", subcore_axis_name="s", num_cores=NC, num_subcores=NS
    ),
    scratch_types=[
        pltpu.VMEM((CH,), jnp.int32),
        pltpu.VMEM((CH,), jnp.int32),
        pltpu.VMEM((CH,), jnp.int32),
        pltpu.VMEM((CH,), jnp.int32),
        pltpu.VMEM((CH, D), jnp.float32),
        pltpu.VMEM((CH, D), jnp.float32),
        pltpu.VMEM((CH, D), jnp.float32),
        pltpu.VMEM((CH, D), jnp.float32),
        pltpu.VMEM((CH,), jnp.float32),
        pltpu.VMEM((2048,), jnp.float32),
        pltpu.VMEM_SHARED((NPAD, D), jnp.float32),
        pltpu.VMEM_SHARED((NPAD,), jnp.float32),
        pltpu.SemaphoreType.DMA,
        pltpu.SemaphoreType.DMA,
        pltpu.SemaphoreType.DMA,
        pltpu.SemaphoreType.DMA,
    ],
)(_sc_body)


def _comb_body(p_ref, deg_ref, x_ref, ws_ref, bs_ref, o_ref):
    ms = p_ref[0] + p_ref[1]
    d = deg_ref[0] + deg_ref[1]
    r = 1.0 / jnp.maximum(d, 1.0)
    sf = (
        jnp.dot(x_ref[...], ws_ref[...], preferred_element_type=jnp.float32)
        + bs_ref[...][None, :]
    )
    o_ref[...] = ms * r + sf


def _combine(msum, deg, x, W_self, b_self):
    return pl.pallas_call(
        _comb_body,
        out_shape=jax.ShapeDtypeStruct((N, D), jnp.float32),
    )(msum, deg, x, W_self, b_self)


def kernel(x, edge_index, edge_attr, W_edge, W_ne, b_ne, W_self, b_self):
    W1 = W_ne[:D, :]
    W2 = W_ne[D:, :]
    xh, W2e = _prep(x, W1, b_ne, W_edge, W2)
    eh2 = _eh2(edge_attr, W2e)
    msum, deg = _sc_call(xh, eh2, edge_index[0], edge_index[1])
    deg2 = deg.reshape(NC, NPAD)[:, :N].reshape(NC, N, 1)
    return _combine(msum[:, :N], deg2, x, W_self, b_self)
